# baseline (speedup 1.0000x reference)
import functools

import jax
import jax.numpy as jnp
import numpy as np
from jax import lax
from jax.experimental import pallas as pl
from jax.experimental.pallas import tpu as pltpu
from jax.experimental.pallas import tpu_sc as plsc

NF = 26
ED = 16
TOT = 104000
B = 4096
NW = 32
PAIRS = 325
ENT = 704
GROUP = 2
GENT = GROUP * ENT
GCH = GENT // 128
GLIN = GROUP * NF
NG = B // GROUP
GPW = NG // NW
NIT = GPW // 2

_OFFSETS = np.arange(NF, dtype=np.int32) * 4000
_II, _JJ = np.triu_indices(NF, 1)
_II = _II.astype(np.int32)
_JJ = _JJ.astype(np.int32)


def _compute_sample(gbuf, lbuf, sbase, lbase):
    accs = [jnp.zeros((ED,), jnp.float32) for _ in range(4)]
    for p in range(PAIRS):
        va = gbuf[sbase + 2 * p]
        vb = gbuf[sbase + 2 * p + 1]
        accs[p % 4] = accs[p % 4] + va * vb
    lacc = jnp.zeros((ED,), jnp.float32)
    for r in range(NF):
        lacc = lacc + lbuf[lbase + r]
    return (accs[0] + accs[1]) + (accs[2] + accs[3]) + lacc


def _sc_body(fidx_hbm, lidx_hbm, bias_hbm, linp_hbm, tab_hbm, out_hbm,
             fidx_a, fidx_b, lidx_a, lidx_b, gbuf_a, gbuf_b, lbuf_a, lbuf_b,
             res_v, bias_v, gsem_a, gsem_b, isem_a, isem_b):
    wid = lax.axis_index("s") * 2 + lax.axis_index("c")
    pltpu.sync_copy(bias_hbm, bias_v)
    bvec = bias_v[...]
    lanes = jnp.arange(16, dtype=jnp.int32)

    def lperm(val, perm):
        return lax.gather(
            val, perm[:, None],
            dimension_numbers=lax.GatherDimensionNumbers(
                offset_dims=(), collapsed_slice_dims=(0,),
                start_index_map=(0,)),
            slice_sizes=(1,),
            mode=lax.GatherScatterMode.PROMISE_IN_BOUNDS)

    def issue_gathers(fidx_v, lidx_v, gbuf, lbuf, gsem):
        for c in range(GCH):
            pltpu.async_copy(tab_hbm.at[fidx_v.at[c]],
                             gbuf.at[pl.ds(c * 128, 128)], gsem)
        pltpu.async_copy(linp_hbm.at[lidx_v], lbuf, gsem)

    def drain_gathers(gbuf, lbuf, gsem):
        for c in range(GCH):
            pltpu.make_async_copy(tab_hbm.at[pl.ds(0, 128)],
                                  gbuf.at[pl.ds(c * 128, 128)], gsem).wait()
        pltpu.make_async_copy(linp_hbm.at[pl.ds(0, GLIN)], lbuf, gsem).wait()

    def issue_idx(g, fidx_v, lidx_v, isem):
        pltpu.async_copy(fidx_hbm.at[g], fidx_v, isem)
        pltpu.async_copy(lidx_hbm.at[g], lidx_v, isem)

    def drain_idx(fidx_v, lidx_v, isem):
        pltpu.make_async_copy(fidx_hbm.at[0], fidx_v, isem).wait()
        pltpu.make_async_copy(lidx_hbm.at[0], lidx_v, isem).wait()

    g0 = wid * GPW
    pltpu.sync_copy(fidx_hbm.at[g0], fidx_a)
    pltpu.sync_copy(lidx_hbm.at[g0], lidx_a)
    issue_gathers(fidx_a, lidx_a, gbuf_a, lbuf_a, gsem_a)
    issue_idx(g0 + 1, fidx_b, lidx_b, isem_b)

    def it_body(t, resvec):
        for gi, (fidx_s, lidx_s, gbuf_s, lbuf_s, gsem_s, isem_s,
                 fidx_o, lidx_o, gbuf_o, lbuf_o, gsem_o, isem_o) in enumerate((
                (fidx_a, lidx_a, gbuf_a, lbuf_a, gsem_a, isem_a,
                 fidx_b, lidx_b, gbuf_b, lbuf_b, gsem_b, isem_b),
                (fidx_b, lidx_b, gbuf_b, lbuf_b, gsem_b, isem_b,
                 fidx_a, lidx_a, gbuf_a, lbuf_a, gsem_a, isem_a))):
            drain_idx(fidx_o, lidx_o, isem_o)
            issue_gathers(fidx_o, lidx_o, gbuf_o, lbuf_o, gsem_o)
            drain_gathers(gbuf_s, lbuf_s, gsem_s)
            issue_idx(g0 + 2 * t + gi + 2, fidx_s, lidx_s, isem_s)
            for s in range(GROUP):
                tot = _compute_sample(gbuf_s, lbuf_s, s * ENT, s * NF)
                for sh in (8, 4, 2, 1):
                    tot = tot + lperm(tot, lanes ^ sh)
                lane_val = (4 * t + 2 * gi + s) & 15
                resvec = resvec + jnp.where(lanes == lane_val, tot, 0.0)
        sig = 1.0 / (1.0 + jnp.exp(-(resvec + bvec)))
        res_v[pl.ds((t // 4) * 16, 16)] = sig
        return jnp.where((t & 3) == 3, jnp.zeros((16,), jnp.float32), resvec)

    lax.fori_loop(0, NIT, it_body, jnp.zeros((16,), jnp.float32))
    drain_gathers(gbuf_a, lbuf_a, gsem_a)
    drain_idx(fidx_b, lidx_b, isem_b)
    pltpu.sync_copy(res_v, out_hbm.at[pl.ds(wid * (B // NW), B // NW)])


@functools.partial(
    pl.kernel,
    mesh=plsc.VectorSubcoreMesh(core_axis_name="c", subcore_axis_name="s"),
    out_type=jax.ShapeDtypeStruct((B,), jnp.float32),
    compiler_params=pltpu.CompilerParams(use_tc_tiling_on_sc=False),
    scratch_types=[
        pltpu.VMEM((GCH, 128), jnp.int32),
        pltpu.VMEM((GCH, 128), jnp.int32),
        pltpu.VMEM((GLIN,), jnp.int32),
        pltpu.VMEM((GLIN,), jnp.int32),
        pltpu.VMEM((GENT, ED), jnp.float32),
        pltpu.VMEM((GENT, ED), jnp.float32),
        pltpu.VMEM((GLIN, ED), jnp.float32),
        pltpu.VMEM((GLIN, ED), jnp.float32),
        pltpu.VMEM((B // NW,), jnp.float32),
        pltpu.VMEM((16,), jnp.float32),
        pltpu.SemaphoreType.DMA,
        pltpu.SemaphoreType.DMA,
        pltpu.SemaphoreType.DMA,
        pltpu.SemaphoreType.DMA,
    ],
)
def _ffm_sc(fidx_hbm, lidx_hbm, bias_hbm, linp_hbm, tab_hbm, out_hbm,
            fidx_a, fidx_b, lidx_a, lidx_b, gbuf_a, gbuf_b, lbuf_a, lbuf_b,
            res_v, bias_v, gsem_a, gsem_b, isem_a, isem_b):
    _sc_body(fidx_hbm, lidx_hbm, bias_hbm, linp_hbm, tab_hbm, out_hbm,
             fidx_a, fidx_b, lidx_a, lidx_b, gbuf_a, gbuf_b, lbuf_a, lbuf_b,
             res_v, bias_v, gsem_a, gsem_b, isem_a, isem_b)


def kernel(x, linear_w, bias, ffm_w):
    idx = (x.astype(jnp.int32) + jnp.asarray(_OFFSETS)[None, :])
    ii = jnp.asarray(_II)
    jj = jnp.asarray(_JJ)
    ea = idx[:, ii] + jj * TOT
    eb = idx[:, jj] + ii * TOT
    ent = jnp.stack([ea, eb], axis=2).reshape(B, 2 * PAIRS)
    ent = jnp.pad(ent, ((0, 0), (0, ENT - 2 * PAIRS)))
    fidx = jnp.pad(ent.reshape(NG, GCH, 128), ((0, 4), (0, 0), (0, 0)))
    lidx = jnp.pad(idx.reshape(NG, GLIN), ((0, 4), (0, 0)))
    bias16 = jnp.broadcast_to(bias.astype(jnp.float32), (16,))
    linp = jnp.pad(linear_w.astype(jnp.float32), ((0, 0), (0, ED - 1)))
    tab = ffm_w.reshape(NF * TOT, ED)
    return _ffm_sc(fidx, lidx, bias16, linp, tab)


# speedup vs baseline: 1.5184x; 1.5184x over previous
import functools

import jax
import jax.numpy as jnp
import numpy as np
from jax import lax
from jax.experimental import pallas as pl
from jax.experimental.pallas import tpu as pltpu
from jax.experimental.pallas import tpu_sc as plsc

NF = 26
ED = 16
NT = 27
ROWF = NT * ED
TOT = 104000
B = 4096
NW = 32
GROUP = 2
GR = GROUP * NF
NG = B // GROUP
GPW = NG // NW
NIT = GPW // 2

_OFFSETS = np.arange(NF, dtype=np.int32) * 4000
_PAIRS = [(i, j) for i in range(NF) for j in range(i + 1, NF)]


def _compute_sample(gbuf, sbase):
    accs = [jnp.zeros((ED,), jnp.float32) for _ in range(4)]
    for p, (i, j) in enumerate(_PAIRS):
        va = gbuf[sbase + i, pl.ds(j * ED, ED)]
        vb = gbuf[sbase + j, pl.ds(i * ED, ED)]
        accs[p % 4] = accs[p % 4] + va * vb
    lacc = jnp.zeros((ED,), jnp.float32)
    for j in range(NF):
        lacc = lacc + gbuf[sbase + j, pl.ds(NF * ED, ED)]
    return (accs[0] + accs[1]) + (accs[2] + accs[3]) + lacc


def _sc_body(sidx_hbm, bias_hbm, wt_hbm, out_hbm,
             idx_a, idx_b, gbuf_a, gbuf_b, res_v, bias_v,
             sem_a, sem_b, isem_a, isem_b):
    wid = lax.axis_index("s") * 2 + lax.axis_index("c")
    pltpu.sync_copy(bias_hbm, bias_v)
    bvec = bias_v[...]
    lanes = jnp.arange(16, dtype=jnp.int32)

    def lperm(val, perm):
        return lax.gather(
            val, perm[:, None],
            dimension_numbers=lax.GatherDimensionNumbers(
                offset_dims=(), collapsed_slice_dims=(0,),
                start_index_map=(0,)),
            slice_sizes=(1,),
            mode=lax.GatherScatterMode.PROMISE_IN_BOUNDS)

    g0 = wid * GPW
    pltpu.sync_copy(sidx_hbm.at[g0], idx_a)
    pltpu.async_copy(wt_hbm.at[idx_a], gbuf_a, sem_a)
    pltpu.async_copy(sidx_hbm.at[g0 + 1], idx_b, isem_b)

    def it_body(t, resvec):
        for gi, (gbuf, gbuf_o, idx_o, sem_o, idx_p, isem_p, sem_w,
                 isem_w) in enumerate((
                (gbuf_a, gbuf_b, idx_b, sem_b, idx_a, isem_a, sem_a, isem_b),
                (gbuf_b, gbuf_a, idx_a, sem_a, idx_b, isem_b, sem_b, isem_a))):
            pltpu.make_async_copy(sidx_hbm.at[g0], idx_o, isem_w).wait()
            pltpu.async_copy(wt_hbm.at[idx_o], gbuf_o, sem_o)
            pltpu.make_async_copy(wt_hbm.at[pl.ds(0, GR)], gbuf, sem_w).wait()
            pltpu.async_copy(sidx_hbm.at[g0 + 2 * t + gi + 2], idx_p, isem_p)
            for s in range(GROUP):
                tot = _compute_sample(gbuf, s * NF)
                for sh in (8, 4, 2, 1):
                    tot = tot + lperm(tot, lanes ^ sh)
                lane_val = (4 * t + 2 * gi + s) & 15
                resvec = resvec + jnp.where(lanes == lane_val, tot, 0.0)
        sig = 1.0 / (1.0 + jnp.exp(-(resvec + bvec)))
        res_v[pl.ds((t // 4) * 16, 16)] = sig
        return jnp.where((t & 3) == 3, jnp.zeros((16,), jnp.float32), resvec)

    lax.fori_loop(0, NIT, it_body, jnp.zeros((16,), jnp.float32))
    pltpu.make_async_copy(wt_hbm.at[pl.ds(0, GR)], gbuf_a, sem_a).wait()
    pltpu.make_async_copy(sidx_hbm.at[g0], idx_b, isem_b).wait()
    pltpu.sync_copy(res_v, out_hbm.at[pl.ds(wid * (B // NW), B // NW)])


@functools.partial(
    pl.kernel,
    mesh=plsc.VectorSubcoreMesh(core_axis_name="c", subcore_axis_name="s"),
    out_type=jax.ShapeDtypeStruct((B,), jnp.float32),
    compiler_params=pltpu.CompilerParams(use_tc_tiling_on_sc=False),
    scratch_types=[
        pltpu.VMEM((GR,), jnp.int32),
        pltpu.VMEM((GR,), jnp.int32),
        pltpu.VMEM((GR, ROWF), jnp.float32),
        pltpu.VMEM((GR, ROWF), jnp.float32),
        pltpu.VMEM((B // NW,), jnp.float32),
        pltpu.VMEM((16,), jnp.float32),
        pltpu.SemaphoreType.DMA,
        pltpu.SemaphoreType.DMA,
        pltpu.SemaphoreType.DMA,
        pltpu.SemaphoreType.DMA,
    ],
)
def _ffm_sc(sidx_hbm, bias_hbm, wt_hbm, out_hbm,
            idx_a, idx_b, gbuf_a, gbuf_b, res_v, bias_v,
            sem_a, sem_b, isem_a, isem_b):
    _sc_body(sidx_hbm, bias_hbm, wt_hbm, out_hbm,
             idx_a, idx_b, gbuf_a, gbuf_b, res_v, bias_v,
             sem_a, sem_b, isem_a, isem_b)


def kernel(x, linear_w, bias, ffm_w):
    idx = (x.astype(jnp.int32) + jnp.asarray(_OFFSETS)[None, :])
    sidx = jnp.pad(idx.reshape(NG, GR), ((0, 4), (0, 0)))
    bias16 = jnp.broadcast_to(bias.astype(jnp.float32), (16,))
    linp = jnp.pad(linear_w.astype(jnp.float32), ((0, 0), (0, ED - 1)))
    wt = jnp.concatenate(
        [ffm_w.transpose(1, 0, 2), linp[:, None, :]], axis=1
    ).reshape(TOT, ROWF)
    return _ffm_sc(sidx, bias16, wt)
